# Initial kernel scaffold; baseline (speedup 1.0000x reference)
import jax
import jax.numpy as jnp
from jax.experimental import pallas as pl


def kernel(features, indices, table, Wk, bk, Wl, bl, Wp, bp):
    raise NotImplementedError("write your pallas kernel here")


# speedup vs baseline: 1.0663x; 1.0663x over previous
import functools

import jax
import jax.numpy as jnp
from jax import lax
from jax.experimental import pallas as pl
from jax.experimental.pallas import tpu as pltpu
from jax.experimental.pallas import tpu_sc as plsc

VOCAB = 100000
EMB = 128
DFEAT = 512
NCLS = 64
B = 4096

_NC = 2
_NS = 16
_NW = _NC * _NS
_BPW = B // _NW


def _sc_gather(table, idx):
    mesh = plsc.VectorSubcoreMesh(core_axis_name="c", subcore_axis_name="s")

    @functools.partial(
        pl.kernel,
        out_type=jax.ShapeDtypeStruct((B, EMB), jnp.float32),
        mesh=mesh,
        scratch_types=[
            pltpu.VMEM((_BPW,), jnp.int32),
            pltpu.VMEM((_BPW, EMB), jnp.float32),
            pltpu.SemaphoreType.DMA,
        ],
    )
    def gather_kernel(table_hbm, idx_hbm, out_hbm, idx_v, rows_v, sem):
        wid = lax.axis_index("s") * _NC + lax.axis_index("c")
        base = wid * _BPW
        pltpu.sync_copy(idx_hbm.at[pl.ds(base, _BPW)], idx_v)
        pltpu.async_copy(table_hbm.at[idx_v], rows_v, sem).wait()
        pltpu.sync_copy(rows_v, out_hbm.at[pl.ds(base, _BPW)])

    return gather_kernel(table, idx)


def _dense_body(f_ref, e_ref, wk_ref, bk_ref, wl_ref, bl_ref, wp_ref, bp_ref,
                o_ref):
    h_f = jnp.maximum(
        jnp.dot(f_ref[...], wk_ref[...], preferred_element_type=jnp.float32)
        + bk_ref[...], 0.0)
    h_e = jnp.maximum(
        jnp.dot(e_ref[...], wl_ref[...], preferred_element_type=jnp.float32)
        + bl_ref[...], 0.0)
    logits = (
        jnp.dot(h_f, wp_ref[:NCLS, :], preferred_element_type=jnp.float32)
        + jnp.dot(h_e, wp_ref[NCLS:, :], preferred_element_type=jnp.float32)
        + bp_ref[...])
    m = jnp.max(logits, axis=-1, keepdims=True)
    p = jnp.exp(logits - m)
    o_ref[...] = p / jnp.sum(p, axis=-1, keepdims=True)


def _tc_dense(features, embs, Wk, bk, Wl, bl, Wp, bp):
    blk = 512
    grid = (B // blk,)
    return pl.pallas_call(
        _dense_body,
        grid=grid,
        in_specs=[
            pl.BlockSpec((blk, DFEAT), lambda i: (i, 0)),
            pl.BlockSpec((blk, EMB), lambda i: (i, 0)),
            pl.BlockSpec((DFEAT, NCLS), lambda i: (0, 0)),
            pl.BlockSpec((1, NCLS), lambda i: (0, 0)),
            pl.BlockSpec((EMB, NCLS), lambda i: (0, 0)),
            pl.BlockSpec((1, NCLS), lambda i: (0, 0)),
            pl.BlockSpec((2 * NCLS, NCLS), lambda i: (0, 0)),
            pl.BlockSpec((1, NCLS), lambda i: (0, 0)),
        ],
        out_specs=pl.BlockSpec((blk, NCLS), lambda i: (i, 0)),
        out_shape=jax.ShapeDtypeStruct((B, NCLS), jnp.float32),
    )(features, embs, Wk, bk.reshape(1, NCLS), Wl, bl.reshape(1, NCLS),
      Wp, bp.reshape(1, NCLS))


def kernel(features, indices, table, Wk, bk, Wl, bl, Wp, bp):
    embs = _sc_gather(table, indices.astype(jnp.int32))
    return _tc_dense(features, embs, Wk, bk, Wl, bl, Wp, bp)
